# Initial kernel scaffold; baseline (speedup 1.0000x reference)
import jax
import jax.numpy as jnp
from jax.experimental import pallas as pl


def kernel(t, P):
    raise NotImplementedError("write your pallas kernel here")


# speedup vs baseline: 1.9687x; 1.9687x over previous
import functools

import jax
import jax.numpy as jnp
from jax import lax
from jax.experimental import pallas as pl
from jax.experimental.pallas import tpu as pltpu
from jax.experimental.pallas import tpu_sc as plsc

NUM_HIDDENS = 1024
MAX_STEP = 8192
NC = 2
NS = 16
NW = NC * NS
CHUNK = 64


@functools.partial(jax.jit, static_argnames=("b_total",))
def _sc_gather(t_flat, P2d, b_total):
    b_per_w = b_total // NW
    nchunk = b_per_w // CHUNK
    mesh = plsc.VectorSubcoreMesh(core_axis_name="c", subcore_axis_name="s")

    @functools.partial(
        pl.kernel,
        mesh=mesh,
        out_type=jax.ShapeDtypeStruct((b_total, NUM_HIDDENS), jnp.float32),
        scratch_types=[
            pltpu.VMEM((b_per_w,), jnp.int32),
            pltpu.VMEM((CHUNK, NUM_HIDDENS), jnp.float32),
            pltpu.SemaphoreType.DMA,
        ],
    )
    def k(t_hbm, P_hbm, out_hbm, idx_v, buf, sem):
        wid = lax.axis_index("s") * NC + lax.axis_index("c")
        base = wid * b_per_w
        pltpu.sync_copy(t_hbm.at[pl.ds(base, b_per_w)], idx_v)
        for c in range(nchunk):
            pltpu.async_copy(
                P_hbm.at[idx_v.at[pl.ds(c * CHUNK, CHUNK)]], buf, sem
            ).wait()
            pltpu.sync_copy(buf, out_hbm.at[pl.ds(base + c * CHUNK, CHUNK)])

    return k(t_flat, P2d)


def kernel(t, P):
    B, S = t.shape
    t_flat = t.reshape(-1)
    P2d = P.reshape(MAX_STEP, NUM_HIDDENS)
    out = _sc_gather(t_flat, P2d, B * S)
    return out.reshape(1, B, S, NUM_HIDDENS)


# speedup vs baseline: 2.0538x; 1.0433x over previous
import functools

import jax
import jax.numpy as jnp
from jax import lax
from jax.experimental import pallas as pl
from jax.experimental.pallas import tpu as pltpu
from jax.experimental.pallas import tpu_sc as plsc

NUM_HIDDENS = 1024
MAX_STEP = 8192
NC = 2
NS = 16
NW = NC * NS
CHUNK = 32
NBUF = 2


@functools.partial(jax.jit, static_argnames=("b_total",))
def _sc_gather(t_flat, P2d, b_total):
    b_per_w = b_total // NW
    nchunk = b_per_w // CHUNK
    mesh = plsc.VectorSubcoreMesh(core_axis_name="c", subcore_axis_name="s")

    @functools.partial(
        pl.kernel,
        mesh=mesh,
        out_type=jax.ShapeDtypeStruct((b_total, NUM_HIDDENS), jnp.float32),
        scratch_types=[
            pltpu.VMEM((b_per_w,), jnp.int32),
            pltpu.VMEM((NBUF, CHUNK, NUM_HIDDENS), jnp.float32),
            pltpu.SemaphoreType.DMA((NBUF,)),
            pltpu.SemaphoreType.DMA((NBUF,)),
        ],
    )
    def k(t_hbm, P_hbm, out_hbm, idx_v, bufs, gsem, osem):
        wid = lax.axis_index("s") * NC + lax.axis_index("c")
        base = wid * b_per_w
        pltpu.sync_copy(t_hbm.at[pl.ds(base, b_per_w)], idx_v)

        def gather(c, p):
            return pltpu.async_copy(
                P_hbm.at[idx_v.at[pl.ds(c * CHUNK, CHUNK)]],
                bufs.at[p],
                gsem.at[p],
            )

        gdesc = [gather(0, 0), None]
        odesc = [None, None]
        for c in range(nchunk):
            p = c % NBUF
            q = (c + 1) % NBUF
            if c + 1 < nchunk:
                if odesc[q] is not None:
                    odesc[q].wait()
                    odesc[q] = None
                gdesc[q] = gather(c + 1, q)
            gdesc[p].wait()
            odesc[p] = pltpu.async_copy(
                bufs.at[p], out_hbm.at[pl.ds(base + c * CHUNK, CHUNK)], osem.at[p]
            )
        for d in odesc:
            if d is not None:
                d.wait()

    return k(t_flat, P2d)


def kernel(t, P):
    B, S = t.shape
    t_flat = t.reshape(-1)
    P2d = P.reshape(MAX_STEP, NUM_HIDDENS)
    out = _sc_gather(t_flat, P2d, B * S)
    return out.reshape(1, B, S, NUM_HIDDENS)


# speedup vs baseline: 2.0814x; 1.0134x over previous
import functools

import jax
import jax.numpy as jnp
from jax import lax
from jax.experimental import pallas as pl
from jax.experimental.pallas import tpu as pltpu
from jax.experimental.pallas import tpu_sc as plsc

NUM_HIDDENS = 1024
MAX_STEP = 8192
NC = 2
NS = 16
NW = NC * NS
CHUNK = 32
NBUF = 3


@functools.partial(jax.jit, static_argnames=("b_total",))
def _sc_gather(t_flat, P2d, b_total):
    b_per_w = b_total // NW
    nchunk = b_per_w // CHUNK
    mesh = plsc.VectorSubcoreMesh(core_axis_name="c", subcore_axis_name="s")

    @functools.partial(
        pl.kernel,
        mesh=mesh,
        out_type=jax.ShapeDtypeStruct((b_total, NUM_HIDDENS), jnp.float32),
        scratch_types=[
            pltpu.VMEM((b_per_w,), jnp.int32),
            pltpu.VMEM((NBUF, CHUNK, NUM_HIDDENS), jnp.float32),
            pltpu.SemaphoreType.DMA((NBUF,)),
            pltpu.SemaphoreType.DMA((NBUF,)),
        ],
    )
    def k(t_hbm, P_hbm, out_hbm, idx_v, bufs, gsem, osem):
        wid = lax.axis_index("s") * NC + lax.axis_index("c")
        base = wid * b_per_w
        pltpu.sync_copy(t_hbm.at[pl.ds(base, b_per_w)], idx_v)

        def gather(c, p):
            return pltpu.async_copy(
                P_hbm.at[idx_v.at[pl.ds(c * CHUNK, CHUNK)]],
                bufs.at[p],
                gsem.at[p],
            )

        depth = NBUF - 1
        gdesc = [None] * NBUF
        odesc = [None] * NBUF
        for c in range(min(depth, nchunk)):
            gdesc[c % NBUF] = gather(c, c % NBUF)
        for c in range(nchunk):
            p = c % NBUF
            nxt = c + depth
            if nxt < nchunk:
                q = nxt % NBUF
                if odesc[q] is not None:
                    odesc[q].wait()
                    odesc[q] = None
                gdesc[q] = gather(nxt, q)
            gdesc[p].wait()
            odesc[p] = pltpu.async_copy(
                bufs.at[p], out_hbm.at[pl.ds(base + c * CHUNK, CHUNK)], osem.at[p]
            )
        for d in odesc:
            if d is not None:
                d.wait()

    return k(t_flat, P2d)


def kernel(t, P):
    B, S = t.shape
    t_flat = t.reshape(-1)
    P2d = P.reshape(MAX_STEP, NUM_HIDDENS)
    out = _sc_gather(t_flat, P2d, B * S)
    return out.reshape(1, B, S, NUM_HIDDENS)


# speedup vs baseline: 2.0817x; 1.0002x over previous
import functools

import jax
import jax.numpy as jnp
from jax import lax
from jax.experimental import pallas as pl
from jax.experimental.pallas import tpu as pltpu
from jax.experimental.pallas import tpu_sc as plsc

NUM_HIDDENS = 1024
MAX_STEP = 8192
NC = 2
NS = 16
NW = NC * NS
CHUNK = 32
NBUF = 3


@functools.partial(jax.jit, static_argnames=("b", "s"))
def _sc_gather(t, P, b, s):
    b_total = b * s
    b_per_w = b_total // NW
    w_per_row = s // b_per_w
    nchunk = b_per_w // CHUNK
    mesh = plsc.VectorSubcoreMesh(core_axis_name="c", subcore_axis_name="s")

    @functools.partial(
        pl.kernel,
        mesh=mesh,
        out_type=jax.ShapeDtypeStruct((1, b, s, NUM_HIDDENS), jnp.float32),
        scratch_types=[
            pltpu.VMEM((b_per_w,), jnp.int32),
            pltpu.VMEM((NBUF, CHUNK, NUM_HIDDENS), jnp.float32),
            pltpu.SemaphoreType.DMA((NBUF,)),
            pltpu.SemaphoreType.DMA((NBUF,)),
        ],
    )
    def k(t_hbm, P_hbm, out_hbm, idx_v, bufs, gsem, osem):
        wid = lax.axis_index("s") * NC + lax.axis_index("c")
        row = wid // w_per_row
        col = (wid % w_per_row) * b_per_w
        pltpu.sync_copy(t_hbm.at[row, pl.ds(col, b_per_w)], idx_v)

        def gather(c, p):
            return pltpu.async_copy(
                P_hbm.at[idx_v.at[pl.ds(c * CHUNK, CHUNK)]],
                bufs.at[p],
                gsem.at[p],
            )

        depth = NBUF - 1
        gdesc = [None] * NBUF
        odesc = [None] * NBUF
        for c in range(min(depth, nchunk)):
            gdesc[c % NBUF] = gather(c, c % NBUF)
        for c in range(nchunk):
            p = c % NBUF
            nxt = c + depth
            if nxt < nchunk:
                q = nxt % NBUF
                if odesc[q] is not None:
                    odesc[q].wait()
                    odesc[q] = None
                gdesc[q] = gather(nxt, q)
            gdesc[p].wait()
            odesc[p] = pltpu.async_copy(
                bufs.at[p],
                out_hbm.at[0, row, pl.ds(col + c * CHUNK, CHUNK)],
                osem.at[p],
            )
        for d in odesc:
            if d is not None:
                d.wait()

    return k(t, P)


def kernel(t, P):
    B, S = t.shape
    P2d = P.reshape(MAX_STEP, NUM_HIDDENS)
    return _sc_gather(t, P2d, B, S)
